# baseline (speedup 1.0000x reference)
import functools

import jax
import jax.numpy as jnp
from jax.experimental import pallas as pl
from jax.experimental.pallas import tpu as pltpu
from jax.experimental.pallas import tpu_sc as plsc

_CHUNK = 64


def kernel(x, pe_weight):
    batch = x.shape[0]
    maxlen, d = pe_weight.shape
    info = plsc.get_sparse_core_info()
    nw = info.num_cores * info.num_subcores
    rows_per_w = maxlen // nw
    nchunk = rows_per_w // _CHUNK
    mesh = plsc.VectorSubcoreMesh(core_axis_name="c", subcore_axis_name="s")

    @functools.partial(
        pl.kernel,
        mesh=mesh,
        out_type=jax.ShapeDtypeStruct((batch, maxlen, d), pe_weight.dtype),
        scratch_types=[
            pltpu.VMEM((_CHUNK, d), pe_weight.dtype),
            pltpu.SemaphoreType.DMA,
        ],
    )
    def _sc_copy(w_hbm, out_hbm, buf, sem):
        wid = jax.lax.axis_index("s") * info.num_cores + jax.lax.axis_index("c")
        base = wid * rows_per_w
        for c in range(nchunk):
            rows = base + c * _CHUNK
            pltpu.sync_copy(w_hbm.at[pl.ds(rows, _CHUNK)], buf)
            copies = [
                pltpu.async_copy(buf, out_hbm.at[b, pl.ds(rows, _CHUNK)], sem)
                for b in range(batch)
            ]
            for cp in copies:
                cp.wait()

    return _sc_copy(pe_weight)


# speedup vs baseline: 1.4991x; 1.4991x over previous
import jax
import jax.numpy as jnp
from jax.experimental import pallas as pl
from jax.experimental.pallas import tpu as pltpu

_CHUNK_ROWS = (2048, 2048, 4096)


def kernel(x, pe_weight):
    batch = x.shape[0]
    maxlen, d = pe_weight.shape
    assert sum(_CHUNK_ROWS) == maxlen
    offs = []
    o = 0
    for c in _CHUNK_ROWS:
        offs.append(o)
        o += c
    nchunk = len(_CHUNK_ROWS)

    def _body(w_hbm, out_hbm, buf, in_sems, out_sems):
        for i, (o, c) in enumerate(zip(offs, _CHUNK_ROWS)):
            pltpu.make_async_copy(
                w_hbm.at[pl.ds(o, c)],
                buf.at[pl.ds(o, c)],
                in_sems.at[i],
            ).start()
        for i, (o, c) in enumerate(zip(offs, _CHUNK_ROWS)):
            pltpu.make_async_copy(
                w_hbm.at[pl.ds(o, c)],
                buf.at[pl.ds(o, c)],
                in_sems.at[i],
            ).wait()
            for b in range(batch):
                pltpu.make_async_copy(
                    buf.at[pl.ds(o, c)],
                    out_hbm.at[b, pl.ds(o, c)],
                    out_sems.at[i, b],
                ).start()
        for i, (o, c) in enumerate(zip(offs, _CHUNK_ROWS)):
            for b in range(batch):
                pltpu.make_async_copy(
                    buf.at[pl.ds(o, c)],
                    out_hbm.at[b, pl.ds(o, c)],
                    out_sems.at[i, b],
                ).wait()

    return pl.pallas_call(
        _body,
        in_specs=[pl.BlockSpec(memory_space=pltpu.MemorySpace.HBM)],
        out_specs=pl.BlockSpec(memory_space=pltpu.MemorySpace.HBM),
        out_shape=jax.ShapeDtypeStruct((batch, maxlen, d), pe_weight.dtype),
        scratch_shapes=[
            pltpu.VMEM((maxlen, d), pe_weight.dtype),
            pltpu.SemaphoreType.DMA((nchunk,)),
            pltpu.SemaphoreType.DMA((nchunk, batch)),
        ],
    )(pe_weight)


# speedup vs baseline: 1.5058x; 1.0045x over previous
import jax
import jax.numpy as jnp
from jax.experimental import pallas as pl
from jax.experimental.pallas import tpu as pltpu

_CHUNK_ROWS = (4096, 4096)


def kernel(x, pe_weight):
    batch = x.shape[0]
    maxlen, d = pe_weight.shape
    assert sum(_CHUNK_ROWS) == maxlen
    offs = []
    o = 0
    for c in _CHUNK_ROWS:
        offs.append(o)
        o += c
    nchunk = len(_CHUNK_ROWS)

    def _body(w_hbm, out_hbm, buf, in_sems, out_sems):
        for i, (o, c) in enumerate(zip(offs, _CHUNK_ROWS)):
            pltpu.make_async_copy(
                w_hbm.at[pl.ds(o, c)],
                buf.at[pl.ds(o, c)],
                in_sems.at[i],
            ).start()
        for i, (o, c) in enumerate(zip(offs, _CHUNK_ROWS)):
            pltpu.make_async_copy(
                w_hbm.at[pl.ds(o, c)],
                buf.at[pl.ds(o, c)],
                in_sems.at[i],
            ).wait()
            for b in range(batch):
                pltpu.make_async_copy(
                    buf.at[pl.ds(o, c)],
                    out_hbm.at[b, pl.ds(o, c)],
                    out_sems.at[i, b],
                ).start()
        for i, (o, c) in enumerate(zip(offs, _CHUNK_ROWS)):
            for b in range(batch):
                pltpu.make_async_copy(
                    buf.at[pl.ds(o, c)],
                    out_hbm.at[b, pl.ds(o, c)],
                    out_sems.at[i, b],
                ).wait()

    return pl.pallas_call(
        _body,
        in_specs=[pl.BlockSpec(memory_space=pltpu.MemorySpace.HBM)],
        out_specs=pl.BlockSpec(memory_space=pltpu.MemorySpace.HBM),
        out_shape=jax.ShapeDtypeStruct((batch, maxlen, d), pe_weight.dtype),
        scratch_shapes=[
            pltpu.VMEM((maxlen, d), pe_weight.dtype),
            pltpu.SemaphoreType.DMA((nchunk,)),
            pltpu.SemaphoreType.DMA((nchunk, batch)),
        ],
    )(pe_weight)


# speedup vs baseline: 1.5079x; 1.0014x over previous
import jax
from jax.experimental import pallas as pl
from jax.experimental.pallas import tpu as pltpu

_CHUNK_ROWS = (4096, 4096)


def kernel(x, pe_weight):
    batch = x.shape[0]
    maxlen, d = pe_weight.shape
    assert sum(_CHUNK_ROWS) == maxlen
    offs = []
    o = 0
    for c in _CHUNK_ROWS:
        offs.append(o)
        o += c
    nchunk = len(_CHUNK_ROWS)

    def _body(w_hbm, out_hbm, buf, in_sems, out_sems):
        for i, (o, c) in enumerate(zip(offs, _CHUNK_ROWS)):
            pltpu.make_async_copy(
                w_hbm.at[pl.ds(o, c)],
                buf.at[pl.ds(o, c)],
                in_sems.at[i],
            ).start()
        for i, (o, c) in enumerate(zip(offs, _CHUNK_ROWS)):
            pltpu.make_async_copy(
                w_hbm.at[pl.ds(o, c)],
                buf.at[pl.ds(o, c)],
                in_sems.at[i],
            ).wait()
            for b in range(batch):
                pltpu.make_async_copy(
                    buf.at[pl.ds(o, c)],
                    out_hbm.at[b, pl.ds(o, c)],
                    out_sems.at[i, b],
                ).start()
        for i, (o, c) in enumerate(zip(offs, _CHUNK_ROWS)):
            for b in range(batch):
                pltpu.make_async_copy(
                    buf.at[pl.ds(o, c)],
                    out_hbm.at[b, pl.ds(o, c)],
                    out_sems.at[i, b],
                ).wait()

    return pl.pallas_call(
        _body,
        in_specs=[pl.BlockSpec(memory_space=pltpu.MemorySpace.HBM)],
        out_specs=pl.BlockSpec(memory_space=pltpu.MemorySpace.HBM),
        out_shape=jax.ShapeDtypeStruct((batch, maxlen, d), pe_weight.dtype),
        scratch_shapes=[
            pltpu.VMEM((maxlen, d), pe_weight.dtype),
            pltpu.SemaphoreType.DMA((nchunk,)),
            pltpu.SemaphoreType.DMA((nchunk, batch)),
        ],
    )(pe_weight)
